# baseline (speedup 1.0000x reference)
import functools

import jax
import jax.numpy as jnp
from jax import lax
from jax.experimental import pallas as pl
from jax.experimental.pallas import tpu as pltpu
from jax.experimental.pallas import tpu_sc as plsc

_D = 1024
_F = 4096
_FH = _F // 2
_E = 8
_K = 2
_T = 4096
_A = _T * _K
_B = 128
_NB = _A // _B + _E
_NP = _NB * _B
_TB = 512
_NC = 2
_NS = 16
_NW = _NC * _NS
_L = 16


def _gate_body(x_ref, wg_ref, bg_ref, g_ref, s_ref, xbf_ref):
    xb = x_ref[...]
    logits = jnp.dot(xb, wg_ref[...],
                     preferred_element_type=jnp.float32) + bg_ref[...]
    iota = lax.broadcasted_iota(jnp.int32, logits.shape, 1)
    m1 = jnp.max(logits, axis=1, keepdims=True)
    i1 = jnp.min(jnp.where(logits == m1, iota, _E), axis=1, keepdims=True)
    sel1 = iota == i1
    neg = jnp.float32(float("-inf"))
    l2 = jnp.where(sel1, neg, logits)
    m2 = jnp.max(l2, axis=1, keepdims=True)
    i2 = jnp.min(jnp.where(l2 == m2, iota, _E), axis=1, keepdims=True)
    sel2 = iota == i2
    e21 = jnp.exp(m2 - m1)
    w1 = 1.0 / (1.0 + e21)
    w2 = e21 / (1.0 + e21)
    g_ref[...] = jnp.where(sel1, w1, 0.0) + jnp.where(sel2, w2, 0.0)
    s_ref[...] = (sel1 | sel2).astype(jnp.int32)
    xbf_ref[...] = xb.astype(jnp.bfloat16)


def _gate(x_flat, Wg, bg2d):
    return pl.pallas_call(
        _gate_body,
        grid=(_T // _TB,),
        in_specs=[
            pl.BlockSpec((_TB, _D), lambda i: (i, 0)),
            pl.BlockSpec((_D, _E), lambda i: (0, 0)),
            pl.BlockSpec((1, _E), lambda i: (0, 0)),
        ],
        out_specs=[
            pl.BlockSpec((_TB, _E), lambda i: (i, 0)),
            pl.BlockSpec((_TB, _E), lambda i: (i, 0)),
            pl.BlockSpec((_TB, _D), lambda i: (i, 0)),
        ],
        out_shape=[
            jax.ShapeDtypeStruct((_T, _E), jnp.float32),
            jax.ShapeDtypeStruct((_T, _E), jnp.int32),
            jax.ShapeDtypeStruct((_T, _D), jnp.bfloat16),
        ],
    )(x_flat, Wg, bg2d)


def _ffn_body(be_ref, bv_ref, xs_ref, w1_ref, b1_ref, w2_ref, b2_ref, wc_ref,
              ys_ref, w1b_ref, w2b_ref):
    f = pl.program_id(0)
    b = pl.program_id(1)
    prev = be_ref[jnp.maximum(b - 1, 0)]

    @pl.when(jnp.logical_or(b == 0, be_ref[b] != prev))
    def _cast():
        w1b_ref[...] = w1_ref[0].astype(jnp.bfloat16)
        w2b_ref[...] = w2_ref[0].astype(jnp.bfloat16)

    @pl.when(bv_ref[b] == 1)
    def _compute():
        h = jnp.maximum(
            jnp.dot(xs_ref[...], w1b_ref[...],
                    preferred_element_type=jnp.float32) + b1_ref[0], 0.0)
        hb = h.astype(jnp.bfloat16)
        out = jnp.dot(hb, w2b_ref[...], preferred_element_type=jnp.float32)
        out = jnp.where(f == 1, out + b2_ref[0], out)
        ys_ref[0] = out * wc_ref[...][:, 0:1]


def _ffn(block_expert, block_valid, xs_bf, W1, b1r, W2, b2r, w_mat):
    grid_spec = pltpu.PrefetchScalarGridSpec(
        num_scalar_prefetch=2,
        grid=(2, _NB),
        in_specs=[
            pl.BlockSpec((_B, _D), lambda f, b, be, bv: (b, 0)),
            pl.BlockSpec((1, _D, _FH), lambda f, b, be, bv: (be[b], 0, f)),
            pl.BlockSpec((1, 1, _FH), lambda f, b, be, bv: (be[b], 0, f)),
            pl.BlockSpec((1, _FH, _D), lambda f, b, be, bv: (be[b], f, 0)),
            pl.BlockSpec((1, 1, _D), lambda f, b, be, bv: (be[b], 0, 0)),
            pl.BlockSpec((_B, 128), lambda f, b, be, bv: (b, 0)),
        ],
        out_specs=pl.BlockSpec((1, _B, _D), lambda f, b, be, bv: (f, b, 0)),
        scratch_shapes=[
            pltpu.VMEM((_D, _FH), jnp.bfloat16),
            pltpu.VMEM((_FH, _D), jnp.bfloat16),
        ],
    )
    return pl.pallas_call(
        _ffn_body,
        grid_spec=grid_spec,
        out_shape=jax.ShapeDtypeStruct((2, _NP, _D), jnp.float32),
    )(block_expert, block_valid, xs_bf, W1, b1r, W2, b2r, w_mat)


def _sc_gather(x_i32, src_tok):
    rpw = _NP // _NW
    cs = 96
    nck = rpw // cs
    dh = _D // 2
    mesh = plsc.VectorSubcoreMesh(core_axis_name="c", subcore_axis_name="s")

    @functools.partial(
        pl.kernel, mesh=mesh,
        out_type=jax.ShapeDtypeStruct((_NP, dh), jnp.int32),
        scratch_types=[
            pltpu.VMEM((cs,), jnp.int32),
            pltpu.VMEM((cs,), jnp.int32),
            pltpu.VMEM((cs, dh), jnp.int32),
            pltpu.VMEM((cs, dh), jnp.int32),
            pltpu.SemaphoreType.DMA,
            pltpu.SemaphoreType.DMA,
            pltpu.SemaphoreType.DMA,
            pltpu.SemaphoreType.DMA,
        ],
    )
    def k(x_hbm, idx_hbm, out_hbm, idx0, idx1, rows0, rows1,
          gs0, gs1, os0, os1):
        wid = lax.axis_index("s") * _NC + lax.axis_index("c")
        base = wid * rpw
        idxs = (idx0, idx1)
        rows = (rows0, rows1)
        gsem = (gs0, gs1)
        osem = (os0, os1)

        pltpu.sync_copy(idx_hbm.at[pl.ds(base, cs)], idx0)
        gathers = [pltpu.async_copy(x_hbm.at[idx0], rows0, gs0)]
        outs = [None, None]
        for i in range(nck):
            s = i % 2
            if i + 1 < nck:
                ns = (i + 1) % 2
                pltpu.sync_copy(
                    idx_hbm.at[pl.ds(base + (i + 1) * cs, cs)], idxs[ns])
                if outs[ns] is not None:
                    outs[ns].wait()
                gathers.append(
                    pltpu.async_copy(x_hbm.at[idxs[ns]], rows[ns], gsem[ns]))
            gathers[i].wait()
            outs[s] = pltpu.async_copy(
                rows[s], out_hbm.at[pl.ds(base + i * cs, cs)], osem[s])
        for o in outs:
            if o is not None:
                o.wait()

    return k(x_i32, src_tok)


def _sc_combine(ys_flat, qa, qb, qc, qd):
    tpw = _T // _NW
    cs = 32
    nck = tpw // cs
    mesh = plsc.VectorSubcoreMesh(core_axis_name="c", subcore_axis_name="s")

    @functools.partial(
        pl.kernel, mesh=mesh,
        out_type=jax.ShapeDtypeStruct((_T, _D), jnp.float32),
        scratch_types=[
            pltpu.VMEM((cs,), jnp.int32),
            pltpu.VMEM((cs,), jnp.int32),
            pltpu.VMEM((cs,), jnp.int32),
            pltpu.VMEM((cs,), jnp.int32),
            pltpu.VMEM((cs,), jnp.int32),
            pltpu.VMEM((cs,), jnp.int32),
            pltpu.VMEM((cs,), jnp.int32),
            pltpu.VMEM((cs,), jnp.int32),
            pltpu.VMEM((cs, _D), jnp.float32),
            pltpu.VMEM((cs, _D), jnp.float32),
            pltpu.SemaphoreType.DMA,
            pltpu.SemaphoreType.DMA,
            pltpu.SemaphoreType.DMA,
            pltpu.SemaphoreType.DMA,
            pltpu.SemaphoreType.DMA,
            pltpu.SemaphoreType.DMA,
        ],
    )
    def k(ys_hbm, qa_hbm, qb_hbm, qc_hbm, qd_hbm, out_hbm,
          ia0, ib0, ic0, id0, ia1, ib1, ic1, id1, o0, o1,
          gs0, gs1, as0, as1, os0, os1):
        wid = lax.axis_index("s") * _NC + lax.axis_index("c")
        base = wid * tpw
        idxs = ((ia0, ib0, ic0, id0), (ia1, ib1, ic1, id1))
        obuf = (o0, o1)
        gsem = (gs0, gs1)
        asem = (as0, as1)
        osem = (os0, os1)
        qhbms = (qa_hbm, qb_hbm, qc_hbm, qd_hbm)

        def load_idx(i, s):
            sl = pl.ds(base + i * cs, cs)
            for q_hbm, ib in zip(qhbms, idxs[s]):
                pltpu.sync_copy(q_hbm.at[sl], ib)

        load_idx(0, 0)
        gathers = [pltpu.async_copy(ys_hbm.at[idxs[0][0]], o0, gs0)]
        outs = [None, None]
        for i in range(nck):
            s = i % 2
            if i + 1 < nck:
                ns = (i + 1) % 2
                load_idx(i + 1, ns)
                if outs[ns] is not None:
                    outs[ns].wait()
                gathers.append(
                    pltpu.async_copy(ys_hbm.at[idxs[ns][0]], obuf[ns],
                                     gsem[ns]))
            gathers[i].wait()
            adds = [pltpu.async_copy(ys_hbm.at[idxs[s][j]], obuf[s],
                                     asem[s], add=True)
                    for j in (1, 2, 3)]
            for a in adds:
                a.wait()
            outs[s] = pltpu.async_copy(
                obuf[s], out_hbm.at[pl.ds(base + i * cs, cs)], osem[s])
        for o in outs:
            if o is not None:
                o.wait()

    return k(ys_flat, qa, qb, qc, qd)


def _dispatch_metadata(G, S):
    a_idx = jnp.nonzero(S.reshape(-1) != 0, size=_A, fill_value=0)[0]
    a_idx = a_idx.astype(jnp.int32)
    tok = a_idx // _E
    expert = a_idx % _E
    wgt = G.reshape(-1)[a_idx]
    order = jnp.argsort(expert).astype(jnp.int32)
    tok_s = tok[order]
    wgt_s = wgt[order]
    exp_s = expert[order]
    counts = jnp.bincount(expert, length=_E).astype(jnp.int32)
    nb_e = (counts + _B - 1) // _B
    pad_start = (jnp.concatenate([jnp.zeros(1, jnp.int32),
                                  jnp.cumsum(nb_e)])[:_E] * _B)
    offs = jnp.concatenate([jnp.zeros(1, jnp.int32),
                            jnp.cumsum(counts)])[:_E]
    rank = jnp.arange(_A, dtype=jnp.int32) - offs[exp_s]
    dst = (pad_start[exp_s] + rank).astype(jnp.int32)
    src_tok = jnp.zeros((_NP,), jnp.int32).at[dst].set(tok_s)
    w_row = jnp.zeros((_NP,), jnp.float32).at[dst].set(wgt_s)
    blk_cum = jnp.cumsum(nb_e)
    total_blocks = blk_cum[-1]
    block_ids = jnp.arange(_NB, dtype=jnp.int32)
    block_expert = jnp.searchsorted(blk_cum, block_ids, side="right")
    block_expert = jnp.clip(block_expert, 0, _E - 1).astype(jnp.int32)
    block_valid = (block_ids < total_blocks).astype(jnp.int32)
    pos = jnp.zeros((_A,), jnp.int32).at[order].set(dst)
    posk = pos.reshape(_T, _K)
    return src_tok, w_row, block_expert, block_valid, posk[:, 0], posk[:, 1]


def kernel(x, Wg, bg, W1, b1, W2, b2):
    x_flat = x.reshape(_T, _D)
    G, S, xbf = _gate(x_flat, Wg, bg.reshape(1, _E))
    src_tok, w_row, block_expert, block_valid, p0, p1 = _dispatch_metadata(G, S)
    x_i32 = lax.bitcast_convert_type(
        xbf.reshape(_T, _D // 2, 2), jnp.int32)
    xs_i32 = _sc_gather(x_i32, src_tok)
    xs_bf = lax.bitcast_convert_type(xs_i32, jnp.bfloat16).reshape(_NP, _D)
    w_mat = jnp.broadcast_to(w_row[:, None], (_NP, 128))
    ys = _ffn(block_expert, block_valid, xs_bf, W1,
              b1.reshape(_E, 1, _F), W2, b2.reshape(_E, 1, _D), w_mat)
    out = _sc_combine(ys.reshape(2 * _NP, _D), p0, p0 + _NP, p1, p1 + _NP)
    return out.reshape(x.shape)
